# baseline (speedup 1.0000x reference)
import jax
import jax.numpy as jnp
from jax.experimental import pallas as pl
from jax.experimental.pallas import tpu as pltpu

_B = 128
_V = 100000
_S = 8
_NS = _B // _S


def _strip_kernel(x_ref, t_ref, rmax_ref, tval_ref):
    x = x_ref[...]
    cols = jax.lax.broadcasted_iota(jnp.int32, (_S, _V), 1)
    is_t = cols == t_ref[...]
    nontgt = jnp.where(is_t, -jnp.inf, x)
    rmax_ref[...] = jnp.maximum(jnp.max(nontgt, axis=1, keepdims=True), 0.0)
    tval_ref[...] = jnp.sum(jnp.where(is_t, x, 0.0), axis=1, keepdims=True)


def _combine_kernel(rmax_ref, tval_ref, o_ref):
    o_ref[0, 0] = jnp.sum(tval_ref[...] - rmax_ref[...]) / _B


@jax.jit
def _run(output, t):
    rmax, tval = pl.pallas_call(
        _strip_kernel,
        grid=(_NS,),
        in_specs=[
            pl.BlockSpec((_S, _V), lambda j: (j, 0)),
            pl.BlockSpec((_S, 1), lambda j: (j, 0)),
        ],
        out_specs=[
            pl.BlockSpec((_S, 1), lambda j: (j, 0)),
            pl.BlockSpec((_S, 1), lambda j: (j, 0)),
        ],
        out_shape=[
            jax.ShapeDtypeStruct((_B, 1), jnp.float32),
            jax.ShapeDtypeStruct((_B, 1), jnp.float32),
        ],
        compiler_params=pltpu.CompilerParams(
            dimension_semantics=("parallel",),
        ),
    )(output, t)
    return pl.pallas_call(
        _combine_kernel,
        out_specs=pl.BlockSpec(memory_space=pltpu.SMEM),
        out_shape=jax.ShapeDtypeStruct((1, 1), jnp.float32),
    )(rmax, tval)


def kernel(output, targetC):
    t = targetC.astype(jnp.int32).reshape(_B, 1)
    return _run(output, t)[0, 0]


# speedup vs baseline: 1.0825x; 1.0825x over previous
import jax
import jax.numpy as jnp
from jax.experimental import pallas as pl
from jax.experimental.pallas import tpu as pltpu

_B = 128
_V = 100000
_S = 16
_NS = _B // _S


def _strip_kernel(x_ref, t_ref, rmax_ref, tval_ref):
    x = x_ref[...]
    cols = jax.lax.broadcasted_iota(jnp.int32, (_S, _V), 1)
    is_t = cols == t_ref[...]
    nontgt = jnp.where(is_t, -jnp.inf, x)
    rmax_ref[...] = jnp.maximum(jnp.max(nontgt, axis=1, keepdims=True), 0.0)
    tval_ref[...] = jnp.sum(jnp.where(is_t, x, 0.0), axis=1, keepdims=True)


def _combine_kernel(rmax_ref, tval_ref, o_ref):
    o_ref[0, 0] = jnp.sum(tval_ref[...] - rmax_ref[...]) / _B


@jax.jit
def _run(output, t):
    rmax, tval = pl.pallas_call(
        _strip_kernel,
        grid=(_NS,),
        in_specs=[
            pl.BlockSpec((_S, _V), lambda j: (j, 0)),
            pl.BlockSpec((_S, 1), lambda j: (j, 0)),
        ],
        out_specs=[
            pl.BlockSpec((_S, 1), lambda j: (j, 0)),
            pl.BlockSpec((_S, 1), lambda j: (j, 0)),
        ],
        out_shape=[
            jax.ShapeDtypeStruct((_B, 1), jnp.float32),
            jax.ShapeDtypeStruct((_B, 1), jnp.float32),
        ],
        compiler_params=pltpu.CompilerParams(
            dimension_semantics=("parallel",),
        ),
    )(output, t)
    return pl.pallas_call(
        _combine_kernel,
        out_specs=pl.BlockSpec(memory_space=pltpu.SMEM),
        out_shape=jax.ShapeDtypeStruct((1, 1), jnp.float32),
    )(rmax, tval)


def kernel(output, targetC):
    t = targetC.astype(jnp.int32).reshape(_B, 1)
    return _run(output, t)[0, 0]


# speedup vs baseline: 1.1033x; 1.0192x over previous
import jax
import jax.numpy as jnp
from jax.experimental import pallas as pl
from jax.experimental.pallas import tpu as pltpu

_B = 128
_V = 100000
_S = 32
_NS = _B // _S


def _strip_kernel(x_ref, t_ref, rmax_ref, tval_ref):
    x = x_ref[...]
    cols = jax.lax.broadcasted_iota(jnp.int32, (_S, _V), 1)
    is_t = cols == t_ref[...]
    nontgt = jnp.where(is_t, -jnp.inf, x)
    rmax_ref[...] = jnp.maximum(jnp.max(nontgt, axis=1, keepdims=True), 0.0)
    tval_ref[...] = jnp.sum(jnp.where(is_t, x, 0.0), axis=1, keepdims=True)


def _combine_kernel(rmax_ref, tval_ref, o_ref):
    o_ref[0, 0] = jnp.sum(tval_ref[...] - rmax_ref[...]) / _B


@jax.jit
def _run(output, t):
    rmax, tval = pl.pallas_call(
        _strip_kernel,
        grid=(_NS,),
        in_specs=[
            pl.BlockSpec((_S, _V), lambda j: (j, 0)),
            pl.BlockSpec((_S, 1), lambda j: (j, 0)),
        ],
        out_specs=[
            pl.BlockSpec((_S, 1), lambda j: (j, 0)),
            pl.BlockSpec((_S, 1), lambda j: (j, 0)),
        ],
        out_shape=[
            jax.ShapeDtypeStruct((_B, 1), jnp.float32),
            jax.ShapeDtypeStruct((_B, 1), jnp.float32),
        ],
        compiler_params=pltpu.CompilerParams(
            dimension_semantics=("parallel",),
        ),
    )(output, t)
    return pl.pallas_call(
        _combine_kernel,
        out_specs=pl.BlockSpec(memory_space=pltpu.SMEM),
        out_shape=jax.ShapeDtypeStruct((1, 1), jnp.float32),
    )(rmax, tval)


def kernel(output, targetC):
    t = targetC.astype(jnp.int32).reshape(_B, 1)
    return _run(output, t)[0, 0]


# speedup vs baseline: 1.1326x; 1.0266x over previous
import jax
import jax.numpy as jnp
from jax.experimental import pallas as pl
from jax.experimental.pallas import tpu as pltpu

_B = 128
_V = 100000
_S = 16
_NS = _B // _S
_K = 4


def _copy(x_hbm, buf, sems, i):
    k = i % _K
    return pltpu.make_async_copy(
        x_hbm.at[pl.ds(i * _S, _S), :], buf.at[k], sems.at[k]
    )


def _stream_kernel(x_hbm, t_ref, o_ref, buf, sems):
    for i in range(_K):
        _copy(x_hbm, buf, sems, i).start()

    cols = jax.lax.broadcasted_iota(jnp.int32, (_S, _V), 1)
    acc = jnp.float32(0.0)
    for i in range(_NS):
        _copy(x_hbm, buf, sems, i).wait()
        x = buf[i % _K]
        is_t = cols == t_ref[pl.ds(i * _S, _S), :]
        rmax = jnp.maximum(jnp.max(jnp.where(is_t, -jnp.inf, x), axis=1), 0.0)
        tval = jnp.sum(jnp.where(is_t, x, 0.0), axis=1)
        acc = acc + jnp.sum(tval - rmax)
        if i + _K < _NS:
            _copy(x_hbm, buf, sems, i + _K).start()
    o_ref[0, 0] = acc / _B


@jax.jit
def _run(output, t):
    return pl.pallas_call(
        _stream_kernel,
        in_specs=[
            pl.BlockSpec(memory_space=pl.ANY),
            pl.BlockSpec(memory_space=pltpu.MemorySpace.VMEM),
        ],
        out_specs=pl.BlockSpec(memory_space=pltpu.SMEM),
        out_shape=jax.ShapeDtypeStruct((1, 1), jnp.float32),
        scratch_shapes=[
            pltpu.VMEM((_K, _S, _V), jnp.float32),
            pltpu.SemaphoreType.DMA((_K,)),
        ],
    )(output, t)


def kernel(output, targetC):
    t = targetC.astype(jnp.int32).reshape(_B, 1)
    return _run(output, t)[0, 0]
